# Initial kernel scaffold; baseline (speedup 1.0000x reference)
import jax
import jax.numpy as jnp
from jax.experimental import pallas as pl


def kernel(features, locations, batch_ids, roi_boxes):
    raise NotImplementedError("write your pallas kernel here")


# speedup vs baseline: 4.9600x; 4.9600x over previous
import functools

import jax
import jax.numpy as jnp
from jax import lax
from jax.experimental import pallas as pl
from jax.experimental.pallas import tpu as pltpu
from jax.experimental.pallas import tpu_sc as plsc

N = 200000
D = 128
R = 64
OUT = 8
NBINS = OUT * OUT
NC = 2
NS = 16
NW = NC * NS
ROIS_PER_W = R // NW
ROWS_PER_W = ROIS_PER_W * NBINS

CHUNK = 2000
NVEC = CHUNK // 16
HITCAP = 2 * CHUNK + 224
GB = 128
NEG = -3.0e38


def _roi_body(feat_hbm, xs_hbm, ys_hbm, bs_hbm, roi_hbm, out_hbm,
              xbuf, ybuf, bbuf, roibuf, hs_buf, hr_buf, acc, featbuf, sem):
    cid = lax.axis_index("c")
    sid = lax.axis_index("s")
    wid = sid * NC + cid
    lanes = lax.iota(jnp.int32, 16)

    def init_body(i, c):
        acc[pl.ds(i * 16, 16)] = jnp.full((16,), NEG, jnp.float32)
        return c
    lax.fori_loop(0, (ROWS_PER_W + 1) * D // 16, init_body, 0)

    pltpu.sync_copy(roi_hbm, roibuf)

    params = []
    for a in range(ROIS_PER_W):
        r = wid * ROIS_PER_W + a
        def bcast(p, r=r):
            return plsc.load_gather(roibuf, [jnp.full((16,), p * R, jnp.int32) + r])
        bv = bcast(0)
        x1 = bcast(1)
        y1 = bcast(2)
        ex = bcast(3) - x1
        ey = bcast(4) - y1
        rex = jnp.float32(OUT) / ex.astype(jnp.float32)
        rey = jnp.float32(OUT) / ey.astype(jnp.float32)
        params.append((bv, x1, y1, ex, ey, rex, rey, a))

    def chunk_body(ci, carry):
        base = ci * CHUNK
        pltpu.sync_copy(xs_hbm.at[pl.ds(base, CHUNK)], xbuf)
        pltpu.sync_copy(ys_hbm.at[pl.ds(base, CHUNK)], ybuf)
        pltpu.sync_copy(bs_hbm.at[pl.ds(base, CHUNK)], bbuf)

        def vec_body(v, cnt_vec):
            off = v * 16
            xv = xbuf[pl.ds(off, 16)]
            yv = ybuf[pl.ds(off, 16)]
            bvv = bbuf[pl.ds(off, 16)]
            site = base + off + lanes
            cur = cnt_vec
            for (bv, x1, y1, ex, ey, rex, rey, a) in params:
                dx = xv - x1
                dy = yv - y1
                ins = ((bvv == bv) & (dx >= 0) & (dx < ex)
                       & (dy >= 0) & (dy < ey))
                pc = plsc.all_reduce_population_count(ins)
                cnt_in = cur

                @pl.when(jnp.any(ins))
                def _():
                    qx = (dx.astype(jnp.float32) * rex).astype(jnp.int32)
                    qy = (dy.astype(jnp.float32) * rey).astype(jnp.int32)
                    rx8 = dx * OUT
                    ry8 = dy * OUT
                    qx_ = (qx + (rx8 >= (qx + 1) * ex).astype(jnp.int32)
                           - (qx * ex > rx8).astype(jnp.int32))
                    qy_ = (qy + (ry8 >= (qy + 1) * ey).astype(jnp.int32)
                           - (qy * ey > ry8).astype(jnp.int32))
                    row = a * NBINS + qx_ * OUT + qy_
                    ones = jnp.where(ins, 1, 0).astype(jnp.int32)
                    pos = cnt_in + jnp.cumsum(ones) - ones
                    plsc.store_scatter(hs_buf, [pos], site, mask=ins)
                    plsc.store_scatter(hr_buf, [pos], row, mask=ins)

                cur = cur + pc
            return cur

        cnt_vec = lax.fori_loop(0, NVEC, vec_body,
                                jnp.zeros((16,), jnp.int32))

        for i in range(GB // 16):
            ppos = cnt_vec + (i * 16) + lanes
            plsc.store_scatter(hs_buf, [ppos], jnp.zeros((16,), jnp.int32))
            plsc.store_scatter(hr_buf, [ppos],
                               jnp.full((16,), ROWS_PER_W, jnp.int32))

        cnt = jnp.max(cnt_vec)
        trips = (cnt + (GB - 1)) >> 7

        def trip_body(t, c):
            idx = hs_buf.at[pl.ds(t * GB, GB)]
            pltpu.async_copy(feat_hbm.at[idx], featbuf, sem).wait()

            def group_body(g, c2):
                rowvec = hr_buf[pl.ds(t * GB + g * 16, 16)]
                for j in range(16):
                    rj = jnp.max(jnp.where(lanes == j, rowvec, 0))
                    ab = rj * D
                    frow = jnp.full((16,), g * 16 + j, jnp.int32)
                    for cb in range(D // 16):
                        fv = plsc.load_gather(featbuf, [frow, cb * 16 + lanes])
                        av = acc[pl.ds(ab + cb * 16, 16)]
                        acc[pl.ds(ab + cb * 16, 16)] = jnp.maximum(av, fv)
                return c2
            lax.fori_loop(0, GB // 16, group_body, 0)
            return c
        lax.fori_loop(0, trips, trip_body, 0)
        return carry

    lax.fori_loop(0, N // CHUNK, chunk_body, 0)

    def fin_body(i, c):
        v = acc[pl.ds(i * 16, 16)]
        acc[pl.ds(i * 16, 16)] = jnp.where(v > jnp.float32(-1.0e38), v,
                                           jnp.float32(0.0))
        return c
    lax.fori_loop(0, ROWS_PER_W * D // 16, fin_body, 0)
    pltpu.sync_copy(acc.at[pl.ds(0, ROWS_PER_W * D)],
                    out_hbm.at[pl.ds(wid * ROWS_PER_W * D, ROWS_PER_W * D)])


@jax.jit
def kernel(features, locations, batch_ids, roi_boxes):
    xs = locations[:, 0] + 0
    ys = locations[:, 1] + 0
    roi_flat = roi_boxes.T.reshape(-1)

    mesh = plsc.VectorSubcoreMesh(core_axis_name="c", subcore_axis_name="s",
                                  num_cores=NC, num_subcores=NS)
    run = pl.kernel(
        _roi_body,
        out_type=jax.ShapeDtypeStruct((R * NBINS * D,), jnp.float32),
        mesh=mesh,
        compiler_params=pltpu.CompilerParams(needs_layout_passes=False),
        scratch_types=[
            pltpu.VMEM((CHUNK,), jnp.int32),
            pltpu.VMEM((CHUNK,), jnp.int32),
            pltpu.VMEM((CHUNK,), jnp.int32),
            pltpu.VMEM((5 * R,), jnp.int32),
            pltpu.VMEM((HITCAP,), jnp.int32),
            pltpu.VMEM((HITCAP,), jnp.int32),
            pltpu.VMEM(((ROWS_PER_W + 1) * D,), jnp.float32),
            pltpu.VMEM((GB, D), jnp.float32),
            pltpu.SemaphoreType.DMA,
        ],
    )
    out = run(features, xs, ys, batch_ids, roi_flat)
    return out.reshape(R * NBINS, D)
